# baseline (speedup 1.0000x reference)
import jax
import jax.numpy as jnp
from jax.experimental import pallas as pl
from jax.experimental.pallas import tpu as pltpu

N = 10000
NFEAT = 256
NCLASS = 64
BM = 200
NSTEP = (N // 2) // BM


def _half(a_ref, support_ref, b_ref, o_ref):
    out = (
        jnp.dot(a_ref[...], support_ref[...], preferred_element_type=jnp.float32)
        + b_ref[...]
    )
    shifted = out - jnp.max(out, axis=1, keepdims=True)
    lse = jnp.log(jnp.sum(jnp.exp(shifted), axis=1, keepdims=True))
    o_ref[...] = shifted - lse


def _body(x_ref, adj_top_ref, adj_bot_ref, w_ref, b_ref,
          out_top_ref, out_bot_ref, support_ref):
    @pl.when(pl.program_id(0) == 0)
    def _():
        support_ref[...] = jnp.dot(
            x_ref[...], w_ref[...], preferred_element_type=jnp.float32
        )

    _half(adj_top_ref, support_ref, b_ref, out_top_ref)
    _half(adj_bot_ref, support_ref, b_ref, out_bot_ref)


@jax.jit
def kernel(x, adj, W, b):
    b2 = b.reshape(1, NCLASS)
    top, bot = pl.pallas_call(
        _body,
        grid=(NSTEP,),
        in_specs=[
            pl.BlockSpec((N, NFEAT), lambda i: (0, 0)),
            pl.BlockSpec((BM, N), lambda i: (i, 0)),
            pl.BlockSpec((BM, N), lambda i: (i + NSTEP, 0)),
            pl.BlockSpec((NFEAT, NCLASS), lambda i: (0, 0)),
            pl.BlockSpec((1, NCLASS), lambda i: (0, 0)),
        ],
        out_specs=[
            pl.BlockSpec((BM, NCLASS), lambda i: (i, 0)),
            pl.BlockSpec((BM, NCLASS), lambda i: (i, 0)),
        ],
        out_shape=[
            jax.ShapeDtypeStruct((N // 2, NCLASS), jnp.float32),
            jax.ShapeDtypeStruct((N // 2, NCLASS), jnp.float32),
        ],
        scratch_shapes=[pltpu.VMEM((N, NCLASS), jnp.float32)],
    )(x, adj, adj, W, b2)
    return jnp.concatenate([top, bot], axis=0)
